# Initial kernel scaffold; baseline (speedup 1.0000x reference)
import jax
import jax.numpy as jnp
from jax.experimental import pallas as pl


def kernel(logits, targets):
    raise NotImplementedError("write your pallas kernel here")


# speedup vs baseline: 42.8826x; 42.8826x over previous
import functools

import jax
import jax.numpy as jnp
from jax import lax
from jax.experimental import pallas as pl
from jax.experimental.pallas import tpu as pltpu
from jax.experimental.pallas import tpu_sc as plsc

N = 32 * 512 * 512
KEEP = N // 2
ROWS, COLS = 512, 16384
NW = 32
ROWS_PER_TILE = ROWS // NW
NBINS = 32768
LANES = 16
VREGS_PER_ROW = COLS // LANES


def _loss_body(x_ref, t_ref, o_ref):
    x = x_ref[...]
    t = t_ref[...]
    o_ref[...] = jnp.maximum(x, 0.0) - x * t + jnp.log1p(jnp.exp(-jnp.abs(x)))


def _compute_loss(x2, t2):
    blk = 32
    return pl.pallas_call(
        _loss_body,
        out_shape=jax.ShapeDtypeStruct((ROWS, COLS), jnp.float32),
        grid=(ROWS // blk,),
        in_specs=[pl.BlockSpec((blk, COLS), lambda i: (i, 0)),
                  pl.BlockSpec((blk, COLS), lambda i: (i, 0))],
        out_specs=pl.BlockSpec((blk, COLS), lambda i: (i, 0)),
    )(x2, t2)


def _hist_body(loss_hbm, cnt_hbm, sum_hbm, cnt_v, sum_v, buf_v, sem):
    wid = lax.axis_index("s") * 2 + lax.axis_index("c")
    base_row = wid * ROWS_PER_TILE

    zero_i = jnp.zeros((LANES,), jnp.int32)
    zero_f = jnp.zeros((LANES,), jnp.float32)

    def zero_body(i, c):
        cnt_v[pl.ds(i * LANES, LANES)] = zero_i
        sum_v[pl.ds(i * LANES, LANES)] = zero_f
        return c

    lax.fori_loop(0, NBINS // LANES, zero_body, 0, unroll=8)

    ones_i = jnp.ones((LANES,), jnp.int32)
    cap = jnp.full((LANES,), NBINS - 1, jnp.int32)

    def row_body(r, c):
        pltpu.async_copy(loss_hbm.at[base_row + r], buf_v, sem).wait()

        def vreg_body(i, c2):
            v = buf_v[pl.ds(i * LANES, LANES)]
            bits = plsc.bitcast(v, jnp.int32)
            hi = jnp.minimum(lax.shift_right_logical(bits, 16), cap)
            plsc.addupdate_scatter(cnt_v, [hi], ones_i)
            plsc.addupdate_scatter(sum_v, [hi], v)
            return c2

        return lax.fori_loop(0, VREGS_PER_ROW, vreg_body, c, unroll=4)

    lax.fori_loop(0, ROWS_PER_TILE, row_body, 0)

    pltpu.sync_copy(cnt_v, cnt_hbm.at[wid])
    pltpu.sync_copy(sum_v, sum_hbm.at[wid])


def _histogram(loss2d):
    mesh = plsc.VectorSubcoreMesh(core_axis_name="c", subcore_axis_name="s")
    fn = pl.kernel(
        _hist_body,
        out_type=[jax.ShapeDtypeStruct((NW, NBINS), jnp.int32),
                  jax.ShapeDtypeStruct((NW, NBINS), jnp.float32)],
        mesh=mesh,
        scratch_types=[pltpu.VMEM((NBINS,), jnp.int32),
                       pltpu.VMEM((NBINS,), jnp.float32),
                       pltpu.VMEM((COLS,), jnp.float32),
                       pltpu.SemaphoreType.DMA],
        compiler_params=pltpu.CompilerParams(needs_layout_passes=False),
    )
    return fn(loss2d)


_HR, _HC = NBINS // 128, 128


def _shift_down_lanes(s, sh):
    pad = jnp.zeros((_HR, sh), s.dtype)
    return jnp.concatenate([s[:, sh:], pad], axis=1)


def _shift_down_rows(s, sh):
    pad = jnp.zeros((sh, _HC), s.dtype)
    return jnp.concatenate([s[sh:, :], pad], axis=0)


def _select_body(cnt_ref, sum_ref, out_ref):
    cnt = jnp.sum(cnt_ref[...], axis=0)
    sm = jnp.sum(sum_ref[...], axis=0)

    s = cnt
    sh = 1
    while sh < _HC:
        s = s + _shift_down_lanes(s, sh)
        sh *= 2
    rowtot = jnp.broadcast_to(s[:, 0:1], (_HR, _HC))
    r = rowtot
    sh = 1
    while sh < _HR:
        r = r + _shift_down_rows(r, sh)
        sh *= 2
    a = s + r - rowtot

    iota_i = lax.broadcasted_iota(jnp.int32, (_HR, _HC), 0)
    iota_j = lax.broadcasted_iota(jnp.int32, (_HR, _HC), 1)
    bidx = iota_i * _HC + iota_j

    t_bin = jnp.max(jnp.where(a >= KEEP, bidx, -1))

    above = bidx > t_bin
    at_t = bidx == t_bin
    c_above = jnp.sum(jnp.where(above, cnt, 0))
    sum_above = jnp.sum(jnp.where(above, sm, 0.0))
    cnt_t = jnp.sum(jnp.where(at_t, cnt, 0))
    sum_t = jnp.sum(jnp.where(at_t, sm, 0.0))

    need = (KEEP - c_above).astype(jnp.float32)
    mean_t = sum_t / cnt_t.astype(jnp.float32)
    total = sum_above + need * mean_t
    out_ref[...] = jnp.broadcast_to(total / float(KEEP), (1, 1))


def _select(cnt, sums):
    return pl.pallas_call(
        _select_body,
        out_shape=jax.ShapeDtypeStruct((1, 1), jnp.float32),
    )(cnt.reshape(NW, _HR, _HC), sums.reshape(NW, _HR, _HC))


def kernel(logits, targets):
    x2 = logits.reshape(ROWS, COLS)
    t2 = targets.reshape(ROWS, COLS)
    loss2d = _compute_loss(x2, t2)
    cnt, sums = _histogram(loss2d)
    out = _select(cnt, sums)
    return out[0, 0]


# speedup vs baseline: 44.5628x; 1.0392x over previous
import functools

import jax
import jax.numpy as jnp
from jax import lax
from jax.experimental import pallas as pl
from jax.experimental.pallas import tpu as pltpu
from jax.experimental.pallas import tpu_sc as plsc

N = 32 * 512 * 512
KEEP = N // 2
ROWS, COLS = 512, 16384
NW = 32
ROWS_PER_TILE = ROWS // NW
SHIFT = 17
NBINS = 16384
LANES = 16


def _loss_body(x_ref, t_ref, o_ref):
    x = x_ref[...]
    t = t_ref[...]
    o_ref[...] = jnp.maximum(x, 0.0) - x * t + jnp.log1p(jnp.exp(-jnp.abs(x)))


def _compute_loss(x2, t2):
    blk = 32
    return pl.pallas_call(
        _loss_body,
        out_shape=jax.ShapeDtypeStruct((ROWS, COLS), jnp.float32),
        grid=(ROWS // blk,),
        in_specs=[pl.BlockSpec((blk, COLS), lambda i: (i, 0)),
                  pl.BlockSpec((blk, COLS), lambda i: (i, 0))],
        out_specs=pl.BlockSpec((blk, COLS), lambda i: (i, 0)),
    )(x2, t2)


def _hist_body(loss_hbm, cnt_hbm, sum_hbm,
               cnt0, sum0, cnt1, sum1, buf0, buf1, sem0, sem1):
    wid = lax.axis_index("s") * 2 + lax.axis_index("c")
    base_row = wid * ROWS_PER_TILE

    zero_i = jnp.zeros((LANES,), jnp.int32)
    zero_f = jnp.zeros((LANES,), jnp.float32)

    def zero_body(i, c):
        sl = pl.ds(i * LANES, LANES)
        cnt0[sl] = zero_i
        cnt1[sl] = zero_i
        sum0[sl] = zero_f
        sum1[sl] = zero_f
        return c

    lax.fori_loop(0, NBINS // LANES, zero_body, 0, unroll=8)

    ones_i = jnp.ones((LANES,), jnp.int32)
    cap = jnp.full((LANES,), NBINS - 1, jnp.int32)

    def process(buf):
        def vreg_body(i, c2):
            o = i * (4 * LANES)
            for q, (cv, sv) in enumerate(
                    ((cnt0, sum0), (cnt1, sum1), (cnt0, sum0), (cnt1, sum1))):
                v = buf[pl.ds(o + q * LANES, LANES)]
                bits = plsc.bitcast(v, jnp.int32)
                hi = jnp.minimum(lax.shift_right_logical(bits, SHIFT), cap)
                plsc.addupdate_scatter(cv, [hi], ones_i)
                plsc.addupdate_scatter(sv, [hi], v)
            return c2

        lax.fori_loop(0, COLS // (4 * LANES), vreg_body, 0, unroll=2)

    pltpu.async_copy(loss_hbm.at[base_row], buf0, sem0)
    pltpu.async_copy(loss_hbm.at[base_row + 1], buf1, sem1)

    def outer(g, c):
        row = g * 2
        pltpu.make_async_copy(loss_hbm.at[base_row], buf0, sem0).wait()
        process(buf0)
        nxt0 = jnp.minimum(row + 2, ROWS_PER_TILE - 2)
        pltpu.async_copy(loss_hbm.at[base_row + nxt0], buf0, sem0)
        pltpu.make_async_copy(loss_hbm.at[base_row], buf1, sem1).wait()
        process(buf1)
        nxt1 = jnp.minimum(row + 3, ROWS_PER_TILE - 1)
        pltpu.async_copy(loss_hbm.at[base_row + nxt1], buf1, sem1)
        return c

    lax.fori_loop(0, ROWS_PER_TILE // 2, outer, 0)
    pltpu.make_async_copy(loss_hbm.at[base_row], buf0, sem0).wait()
    pltpu.make_async_copy(loss_hbm.at[base_row], buf1, sem1).wait()

    def merge_body(i, c):
        sl = pl.ds(i * LANES, LANES)
        cnt0[sl] = cnt0[sl] + cnt1[sl]
        sum0[sl] = sum0[sl] + sum1[sl]
        return c

    lax.fori_loop(0, NBINS // LANES, merge_body, 0, unroll=8)

    pltpu.sync_copy(cnt0, cnt_hbm.at[wid])
    pltpu.sync_copy(sum0, sum_hbm.at[wid])


def _histogram(loss2d):
    mesh = plsc.VectorSubcoreMesh(core_axis_name="c", subcore_axis_name="s")
    fn = pl.kernel(
        _hist_body,
        out_type=[jax.ShapeDtypeStruct((NW, NBINS), jnp.int32),
                  jax.ShapeDtypeStruct((NW, NBINS), jnp.float32)],
        mesh=mesh,
        scratch_types=[pltpu.VMEM((NBINS,), jnp.int32),
                       pltpu.VMEM((NBINS,), jnp.float32),
                       pltpu.VMEM((NBINS,), jnp.int32),
                       pltpu.VMEM((NBINS,), jnp.float32),
                       pltpu.VMEM((COLS,), jnp.float32),
                       pltpu.VMEM((COLS,), jnp.float32),
                       pltpu.SemaphoreType.DMA,
                       pltpu.SemaphoreType.DMA],
        compiler_params=pltpu.CompilerParams(needs_layout_passes=False),
    )
    return fn(loss2d)


_HR, _HC = NBINS // 128, 128


def _shift_down_lanes(s, sh):
    pad = jnp.zeros((_HR, sh), s.dtype)
    return jnp.concatenate([s[:, sh:], pad], axis=1)


def _shift_down_rows(s, sh):
    pad = jnp.zeros((sh, _HC), s.dtype)
    return jnp.concatenate([s[sh:, :], pad], axis=0)


def _select_body(cnt_ref, sum_ref, out_ref):
    cnt = jnp.sum(cnt_ref[...], axis=0)
    sm = jnp.sum(sum_ref[...], axis=0)

    s = cnt
    sh = 1
    while sh < _HC:
        s = s + _shift_down_lanes(s, sh)
        sh *= 2
    rowtot = jnp.broadcast_to(s[:, 0:1], (_HR, _HC))
    r = rowtot
    sh = 1
    while sh < _HR:
        r = r + _shift_down_rows(r, sh)
        sh *= 2
    a = s + r - rowtot

    iota_i = lax.broadcasted_iota(jnp.int32, (_HR, _HC), 0)
    iota_j = lax.broadcasted_iota(jnp.int32, (_HR, _HC), 1)
    bidx = iota_i * _HC + iota_j

    t_bin = jnp.max(jnp.where(a >= KEEP, bidx, -1))

    above = bidx > t_bin
    at_t = bidx == t_bin
    c_above = jnp.sum(jnp.where(above, cnt, 0))
    sum_above = jnp.sum(jnp.where(above, sm, 0.0))
    cnt_t = jnp.sum(jnp.where(at_t, cnt, 0))
    sum_t = jnp.sum(jnp.where(at_t, sm, 0.0))

    need = (KEEP - c_above).astype(jnp.float32)
    mean_t = sum_t / cnt_t.astype(jnp.float32)
    total = sum_above + need * mean_t
    out_ref[...] = jnp.broadcast_to(total / float(KEEP), (1, 1))


def _select(cnt, sums):
    return pl.pallas_call(
        _select_body,
        out_shape=jax.ShapeDtypeStruct((1, 1), jnp.float32),
    )(cnt.reshape(NW, _HR, _HC), sums.reshape(NW, _HR, _HC))


def kernel(logits, targets):
    x2 = logits.reshape(ROWS, COLS)
    t2 = targets.reshape(ROWS, COLS)
    loss2d = _compute_loss(x2, t2)
    cnt, sums = _histogram(loss2d)
    out = _select(cnt, sums)
    return out[0, 0]
